# baseline (speedup 1.0000x reference)
import jax
import jax.numpy as jnp
from jax.experimental import pallas as pl

_BIG_I = 2**30


def _chamfer_body(chunk, x1_ref, x2t_ref, idx1_ref, idx2_ref):
    n = x1_ref.shape[1]
    m = x2t_ref.shape[2]
    x1 = x1_ref[0]
    a2 = jnp.sum(x1 * x1, axis=1, keepdims=True)

    def body(c, carry):
        best_val, best_idx = carry
        j0 = c * chunk
        x2c = x2t_ref[0, :, pl.ds(j0, chunk)]
        b2c = jnp.sum(x2c * x2c, axis=0, keepdims=True)
        ab = jax.lax.dot_general(
            x1, x2c, (((1,), (0,)), ((), ())),
            preferred_element_type=jnp.float32)
        d = a2 + b2c - 2.0 * ab
        rmin = jnp.min(d, axis=1, keepdims=True)
        rarg = jnp.argmin(d, axis=1).astype(jnp.int32)[:, None] + j0
        take = rmin < best_val
        best_val = jnp.where(take, rmin, best_val)
        best_idx = jnp.where(take, rarg, best_idx)
        carg = jnp.argmin(d, axis=0).astype(jnp.int32)[None, :]
        idx2_ref[0, :, pl.ds(j0, chunk)] = carg
        return best_val, best_idx

    init = (jnp.full((n, 1), jnp.inf, jnp.float32),
            jnp.zeros((n, 1), jnp.int32))
    _, best_idx = jax.lax.fori_loop(0, m // chunk, body, init)
    idx1_ref[0, :, :] = best_idx.reshape(1, n)


def kernel(xyz1, xyz2):
    b, n, d = xyz1.shape
    m = xyz2.shape[1]
    chunk = 512
    x2t = jnp.transpose(xyz2, (0, 2, 1))
    idx1, idx2 = pl.pallas_call(
        lambda *refs: _chamfer_body(chunk, *refs),
        grid=(b,),
        in_specs=[pl.BlockSpec((1, n, d), lambda i: (i, 0, 0)),
                  pl.BlockSpec((1, d, m), lambda i: (i, 0, 0))],
        out_specs=[pl.BlockSpec((1, 1, n), lambda i: (i, 0, 0)),
                   pl.BlockSpec((1, 1, m), lambda i: (i, 0, 0))],
        out_shape=[jax.ShapeDtypeStruct((b, 1, n), jnp.int32),
                   jax.ShapeDtypeStruct((b, 1, m), jnp.int32)],
    )(xyz1, x2t)
    return idx1.reshape(b, n), idx2.reshape(b, m)


# speedup vs baseline: 1.7623x; 1.7623x over previous
import jax
import jax.numpy as jnp
from jax.experimental import pallas as pl

_BIG_I = 2**30


def _chamfer_body(chunk, x1_ref, x2t_ref, idx1_ref, idx2_ref):
    n = x1_ref.shape[1]
    m = x2t_ref.shape[2]
    x1 = x1_ref[0]
    a2 = jnp.sum(x1 * x1, axis=1, keepdims=True)
    jidx = jax.lax.broadcasted_iota(jnp.int32, (n, chunk), 1)
    iidx = jax.lax.broadcasted_iota(jnp.int32, (n, chunk), 0)

    def body(c, carry):
        best_val, best_idx = carry
        j0 = c * chunk
        x2c = x2t_ref[0, :, pl.ds(j0, chunk)]
        b2c = jnp.sum(x2c * x2c, axis=0, keepdims=True)
        ab = jax.lax.dot_general(
            x1, x2c, (((1,), (0,)), ((), ())),
            preferred_element_type=jnp.float32)
        d = a2 + b2c - 2.0 * ab
        rmin = jnp.min(d, axis=1, keepdims=True)
        rarg = jnp.min(jnp.where(d == rmin, jidx, _BIG_I),
                       axis=1, keepdims=True) + j0
        take = rmin < best_val
        best_val = jnp.where(take, rmin, best_val)
        best_idx = jnp.where(take, rarg, best_idx)
        cmin = jnp.min(d, axis=0, keepdims=True)
        carg = jnp.min(jnp.where(d == cmin, iidx, _BIG_I),
                       axis=0, keepdims=True)
        idx2_ref[0, :, pl.ds(j0, chunk)] = carg
        return best_val, best_idx

    init = (jnp.full((n, 1), jnp.inf, jnp.float32),
            jnp.zeros((n, 1), jnp.int32))
    _, best_idx = jax.lax.fori_loop(0, m // chunk, body, init)
    idx1_ref[0, :, :] = best_idx.reshape(1, n)


def kernel(xyz1, xyz2):
    b, n, d = xyz1.shape
    m = xyz2.shape[1]
    chunk = 512
    x2t = jnp.transpose(xyz2, (0, 2, 1))
    idx1, idx2 = pl.pallas_call(
        lambda *refs: _chamfer_body(chunk, *refs),
        grid=(b,),
        in_specs=[pl.BlockSpec((1, n, d), lambda i: (i, 0, 0)),
                  pl.BlockSpec((1, d, m), lambda i: (i, 0, 0))],
        out_specs=[pl.BlockSpec((1, 1, n), lambda i: (i, 0, 0)),
                   pl.BlockSpec((1, 1, m), lambda i: (i, 0, 0))],
        out_shape=[jax.ShapeDtypeStruct((b, 1, n), jnp.int32),
                   jax.ShapeDtypeStruct((b, 1, m), jnp.int32)],
    )(xyz1, x2t)
    return idx1.reshape(b, n), idx2.reshape(b, m)


# speedup vs baseline: 2.3537x; 1.3356x over previous
import jax
import jax.numpy as jnp
from jax.experimental import pallas as pl


def _chamfer_body(chunk, x1_ref, x2t_ref, idx1_ref, idx2_ref):
    n = x1_ref.shape[1]
    m = x2t_ref.shape[2]
    x1 = x1_ref[0]
    x2t = x2t_ref[0]
    a2 = jnp.sum(x1 * x1, axis=1, keepdims=True)
    b2 = jnp.sum(x2t * x2t, axis=0, keepdims=True)
    x1d = x1 + x1
    jidx = jax.lax.broadcasted_iota(
        jnp.int32, (n, chunk), 1).astype(jnp.float32)
    iidx = jax.lax.broadcasted_iota(
        jnp.int32, (n, chunk), 0).astype(jnp.float32)

    best_val = jnp.full((n, 1), jnp.inf, jnp.float32)
    best_idx = jnp.zeros((n, 1), jnp.float32)
    for c in range(m // chunk):
        j0 = c * chunk
        x2c = x2t[:, j0:j0 + chunk]
        b2c = b2[:, j0:j0 + chunk]
        ab2 = jax.lax.dot_general(
            x1d, x2c, (((1,), (0,)), ((), ())),
            preferred_element_type=jnp.float32)
        d = (a2 + b2c) - ab2
        rmin = jnp.min(d, axis=1, keepdims=True)
        rarg = jnp.min(jnp.where(d == rmin, jidx, jnp.inf),
                       axis=1, keepdims=True) + float(j0)
        take = rmin < best_val
        best_val = jnp.where(take, rmin, best_val)
        best_idx = jnp.where(take, rarg, best_idx)
        cmin = jnp.min(d, axis=0, keepdims=True)
        carg = jnp.min(jnp.where(d == cmin, iidx, jnp.inf),
                       axis=0, keepdims=True)
        idx2_ref[0, :, j0:j0 + chunk] = carg.astype(jnp.int32)

    idx1_ref[0, :, :] = best_idx.astype(jnp.int32).reshape(1, n)


def kernel(xyz1, xyz2):
    b, n, d = xyz1.shape
    m = xyz2.shape[1]
    chunk = 512
    x2t = jnp.transpose(xyz2, (0, 2, 1))
    idx1, idx2 = pl.pallas_call(
        lambda *refs: _chamfer_body(chunk, *refs),
        grid=(b,),
        in_specs=[pl.BlockSpec((1, n, d), lambda i: (i, 0, 0)),
                  pl.BlockSpec((1, d, m), lambda i: (i, 0, 0))],
        out_specs=[pl.BlockSpec((1, 1, n), lambda i: (i, 0, 0)),
                   pl.BlockSpec((1, 1, m), lambda i: (i, 0, 0))],
        out_shape=[jax.ShapeDtypeStruct((b, 1, n), jnp.int32),
                   jax.ShapeDtypeStruct((b, 1, m), jnp.int32)],
    )(xyz1, x2t)
    return idx1.reshape(b, n), idx2.reshape(b, m)
